# baseline (speedup 1.0000x reference)
import jax
import jax.numpy as jnp
from jax import lax
from jax.experimental import pallas as pl
from jax.experimental.pallas import tpu as pltpu
from jax.experimental.pallas import tpu_sc as plsc

N = 16384
NUM_EVENT = 100000
NUM_STATION = 64
NC = 2
NS = 16
L = 16
NW = NC * NS
PW = N // NW
NVEC = PW // L


def _dist_from_sq(d2):
    i = plsc.bitcast(d2, jnp.int32)
    i = jnp.int32(0x5F3759DF) - (i >> 1)
    y = plsc.bitcast(i, jnp.float32)
    h = jnp.float32(0.5) * d2
    for _ in range(3):
        y = y * (jnp.float32(1.5) - h * y * y)
    return jnp.where(d2 > 0.0, d2 * y, jnp.float32(0.0))


def _body(st_idx_hbm, ev_idx_hbm, ptype_hbm, ptime_hbm, pweight_hbm,
          ev_hbm, stloc_hbm, stdt_hbm,
          pred_hbm, resid_hbm, part_hbm,
          ei_v, gi_v, ev_v, si_v, pt_v, ptm_v, pw_v, stloc_v, stdt_v,
          pred_v, resid_v, acc_v, sem):
    wid = lax.axis_index("s") * NC + lax.axis_index("c")
    base = wid * PW

    lane = lax.iota(jnp.int32, L)
    zero = jnp.zeros((L,), jnp.int32)

    pltpu.sync_copy(ev_idx_hbm.at[pl.ds(base, PW)], ei_v)
    for j in range(NVEC):
        o = j * L
        gi_v[pl.ds(o, L)] = ei_v[pl.ds(o, L)] >> 2
    gather = pltpu.async_copy(ev_hbm.at[gi_v], ev_v, sem)

    pltpu.sync_copy(st_idx_hbm.at[pl.ds(base, PW)], si_v)
    pltpu.sync_copy(ptype_hbm.at[pl.ds(base, PW)], pt_v)
    pltpu.sync_copy(ptime_hbm.at[pl.ds(base, PW)], ptm_v)
    pltpu.sync_copy(pweight_hbm.at[pl.ds(base, PW)], pw_v)
    pltpu.sync_copy(stloc_hbm, stloc_v)
    pltpu.sync_copy(stdt_hbm, stdt_v)
    gather.wait()

    acc = jnp.zeros((L,), jnp.float32)
    for j in range(NVEC):
        o = j * L
        pick = lane + o
        off = (ei_v[pl.ds(o, L)] & 3) << 2
        ex = plsc.load_gather(ev_v, [pick, off])
        ey = plsc.load_gather(ev_v, [pick, off + 1])
        ez = plsc.load_gather(ev_v, [pick, off + 2])
        et = plsc.load_gather(ev_v, [pick, off + 3])
        si = si_v[pl.ds(o, L)]
        s3 = si * 3
        sx = plsc.load_gather(stloc_v, [s3])
        sy = plsc.load_gather(stloc_v, [s3 + 1])
        sz = plsc.load_gather(stloc_v, [s3 + 2])
        sd = plsc.load_gather(stdt_v, [si])
        pt = pt_v[pl.ds(o, L)]
        ptm = ptm_v[pl.ds(o, L)]
        pw = pw_v[pl.ds(o, L)]
        dx = ex - sx
        dy = ey - sy
        dz = ez - sz
        dist = _dist_from_sq(dx * dx + dy * dy + dz * dz)
        vel = jnp.where(pt == 0, jnp.float32(6.0), jnp.float32(6.0 / 1.73))
        t = et + dist / vel + sd
        r = ptm - t
        pred_v[pl.ds(o, L)] = t
        resid_v[pl.ds(o, L)] = r
        ae = jnp.abs(r)
        hub = jnp.where(ae <= 1.0, jnp.float32(0.5) * r * r, ae - jnp.float32(0.5))
        acc = acc + hub * pw

    acc_v[...] = acc
    pltpu.sync_copy(pred_v, pred_hbm.at[pl.ds(base, PW)])
    pltpu.sync_copy(resid_v, resid_hbm.at[pl.ds(base, PW)])
    pltpu.sync_copy(acc_v, part_hbm.at[wid])


def kernel(station_index, event_index, phase_type, phase_time, phase_weight,
           event_loc_w, event_time_w, station_loc_w, station_dt_w):
    ev_packed = jnp.concatenate(
        [event_loc_w, event_time_w], axis=1).reshape(NUM_EVENT // 4, 16)
    mesh = plsc.VectorSubcoreMesh(core_axis_name="c", subcore_axis_name="s")
    out_type = [
        jax.ShapeDtypeStruct((N,), jnp.float32),
        jax.ShapeDtypeStruct((N,), jnp.float32),
        jax.ShapeDtypeStruct((NW, L), jnp.float32),
    ]
    scratch = [
        pltpu.VMEM((PW,), jnp.int32),
        pltpu.VMEM((PW,), jnp.int32),
        pltpu.VMEM((PW, L), jnp.float32),
        pltpu.VMEM((PW,), jnp.int32),
        pltpu.VMEM((PW,), jnp.int32),
        pltpu.VMEM((PW,), jnp.float32),
        pltpu.VMEM((PW,), jnp.float32),
        pltpu.VMEM((NUM_STATION * 3,), jnp.float32),
        pltpu.VMEM((NUM_STATION,), jnp.float32),
        pltpu.VMEM((PW,), jnp.float32),
        pltpu.VMEM((PW,), jnp.float32),
        pltpu.VMEM((L,), jnp.float32),
        pltpu.SemaphoreType.DMA,
    ]
    pred, resid, part = pl.kernel(
        _body, out_type=out_type, mesh=mesh, scratch_types=scratch,
        compiler_params=pltpu.CompilerParams(
            needs_layout_passes=False, use_tc_tiling_on_sc=False))(
        station_index, event_index, phase_type, phase_time, phase_weight,
        ev_packed, station_loc_w.reshape(-1), station_dt_w.reshape(-1))
    return (pred, resid, jnp.sum(part))


# speedup vs baseline: 1.2690x; 1.2690x over previous
import jax
import jax.numpy as jnp
from jax import lax
from jax.experimental import pallas as pl
from jax.experimental.pallas import tpu as pltpu
from jax.experimental.pallas import tpu_sc as plsc

N = 16384
NUM_EVENT = 100000
NUM_STATION = 64
NC = 2
NS = 16
L = 16
NW = NC * NS
PW = N // NW
NVEC = PW // L


def _dist_from_sq(d2):
    i = plsc.bitcast(d2, jnp.int32)
    i = jnp.int32(0x5F3759DF) - (i >> 1)
    y = plsc.bitcast(i, jnp.float32)
    h = jnp.float32(0.5) * d2
    for _ in range(3):
        y = y * (jnp.float32(1.5) - h * y * y)
    return jnp.where(d2 > 0.0, d2 * y, jnp.float32(0.0))


def _body(st_idx_hbm, ev_idx_hbm, ptype_hbm, ptime_hbm, pweight_hbm,
          ev_hbm, stloc_hbm, stdt_hbm,
          pred_hbm, resid_hbm, part_hbm,
          ei_v, ev_v, si_v, pt_v, ptm_v, pw_v, stloc_v, stdt_v,
          pred_v, resid_v, acc_v, sem):
    wid = lax.axis_index("s") * NC + lax.axis_index("c")
    base = wid * PW

    lane = lax.iota(jnp.int32, L)
    zero = jnp.zeros((L,), jnp.int32)

    pltpu.sync_copy(ev_idx_hbm.at[pl.ds(base, PW)], ei_v)
    gather = pltpu.async_copy(ev_hbm.at[ei_v], ev_v, sem)

    pltpu.sync_copy(st_idx_hbm.at[pl.ds(base, PW)], si_v)
    pltpu.sync_copy(ptype_hbm.at[pl.ds(base, PW)], pt_v)
    pltpu.sync_copy(ptime_hbm.at[pl.ds(base, PW)], ptm_v)
    pltpu.sync_copy(pweight_hbm.at[pl.ds(base, PW)], pw_v)
    pltpu.sync_copy(stloc_hbm, stloc_v)
    pltpu.sync_copy(stdt_hbm, stdt_v)
    gather.wait()

    acc = jnp.zeros((L,), jnp.float32)
    for j in range(NVEC):
        o = j * L
        pick = lane + o
        ex = plsc.load_gather(ev_v, [pick, zero])
        ey = plsc.load_gather(ev_v, [pick, zero + 1])
        ez = plsc.load_gather(ev_v, [pick, zero + 2])
        et = plsc.load_gather(ev_v, [pick, zero + 3])
        si = si_v[pl.ds(o, L)]
        s3 = si * 3
        sx = plsc.load_gather(stloc_v, [s3])
        sy = plsc.load_gather(stloc_v, [s3 + 1])
        sz = plsc.load_gather(stloc_v, [s3 + 2])
        sd = plsc.load_gather(stdt_v, [si])
        pt = pt_v[pl.ds(o, L)]
        ptm = ptm_v[pl.ds(o, L)]
        pw = pw_v[pl.ds(o, L)]
        dx = ex - sx
        dy = ey - sy
        dz = ez - sz
        dist = _dist_from_sq(dx * dx + dy * dy + dz * dz)
        vel = jnp.where(pt == 0, jnp.float32(6.0), jnp.float32(6.0 / 1.73))
        t = et + dist / vel + sd
        r = ptm - t
        pred_v[pl.ds(o, L)] = t
        resid_v[pl.ds(o, L)] = r
        ae = jnp.abs(r)
        hub = jnp.where(ae <= 1.0, jnp.float32(0.5) * r * r, ae - jnp.float32(0.5))
        acc = acc + hub * pw

    acc_v[...] = acc
    pltpu.sync_copy(pred_v, pred_hbm.at[pl.ds(base, PW)])
    pltpu.sync_copy(resid_v, resid_hbm.at[pl.ds(base, PW)])
    pltpu.sync_copy(acc_v, part_hbm.at[wid])


def kernel(station_index, event_index, phase_type, phase_time, phase_weight,
           event_loc_w, event_time_w, station_loc_w, station_dt_w):
    ev_packed = jnp.concatenate(
        [event_loc_w[:, 0:1], event_loc_w[:, 1:2], event_loc_w[:, 2:3],
         event_time_w, jnp.zeros((NUM_EVENT, 12), jnp.float32)], axis=1)
    mesh = plsc.VectorSubcoreMesh(core_axis_name="c", subcore_axis_name="s")
    out_type = [
        jax.ShapeDtypeStruct((N,), jnp.float32),
        jax.ShapeDtypeStruct((N,), jnp.float32),
        jax.ShapeDtypeStruct((NW, L), jnp.float32),
    ]
    scratch = [
        pltpu.VMEM((PW,), jnp.int32),
        pltpu.VMEM((PW, L), jnp.float32),
        pltpu.VMEM((PW,), jnp.int32),
        pltpu.VMEM((PW,), jnp.int32),
        pltpu.VMEM((PW,), jnp.float32),
        pltpu.VMEM((PW,), jnp.float32),
        pltpu.VMEM((NUM_STATION * 3,), jnp.float32),
        pltpu.VMEM((NUM_STATION,), jnp.float32),
        pltpu.VMEM((PW,), jnp.float32),
        pltpu.VMEM((PW,), jnp.float32),
        pltpu.VMEM((L,), jnp.float32),
        pltpu.SemaphoreType.DMA,
    ]
    pred, resid, part = pl.kernel(
        _body, out_type=out_type, mesh=mesh, scratch_types=scratch,
        compiler_params=pltpu.CompilerParams(
            needs_layout_passes=False, use_tc_tiling_on_sc=False))(
        station_index, event_index, phase_type, phase_time, phase_weight,
        ev_packed, station_loc_w.reshape(-1), station_dt_w.reshape(-1))
    return (pred, resid, jnp.sum(part))
